# baseline (speedup 1.0000x reference)
import functools

import jax
import jax.numpy as jnp
from jax import lax
from jax.experimental import pallas as pl
from jax.experimental.pallas import tpu as pltpu
from jax.experimental.pallas import tpu_sc as plsc

_C = 1.0
_MAX_NORM = 1e6
_EPS = 1e-15
_BALL_EPS = 4e-3
_HALF = 64

_NC = 2
_NS = 16
_NW = _NC * _NS
_L = 16
_K = 64

_NCH0 = 231
_NCH1 = 85


def _artanh(x):
    x = jnp.clip(x, -1.0 + 1e-7, 1.0 - 1e-7)
    return 0.5 * (jnp.log1p(x) - jnp.log1p(-x))


def _pre_body(x_ref, o_ref):
    v = x_ref[...]
    outs = []
    for lo in (0, _HALF):
        s = v[:, lo:lo + _HALF]
        n = jnp.maximum(jnp.sqrt(jnp.sum(s * s, axis=1, keepdims=True)), _EPS)
        outs.append(s * (_artanh(n) / n))
    o_ref[...] = jnp.concatenate(outs, axis=1)


def _post_body(p0_ref, p1_ref, o_ref):
    s = jnp.minimum(p0_ref[0] + p1_ref[0], _MAX_NORM)
    outs = []
    for lo in (0, _HALF):
        u = s[:, lo:lo + _HALF]
        n = jnp.maximum(jnp.sqrt(jnp.sum(u * u, axis=1, keepdims=True)), _EPS)
        y = u * (jnp.tanh(n) / n)
        yn = jnp.maximum(jnp.sqrt(jnp.sum(y * y, axis=1, keepdims=True)), _EPS)
        maxn = 1.0 - _BALL_EPS
        outs.append(jnp.where(yn > maxn, y / yn * maxn, y))
    o_ref[...] = jnp.concatenate(outs, axis=1)


@functools.partial(jax.jit, static_argnums=(1, 2))
def _sc_spmm(args, n_pad, d):
    rows_per_tile = n_pad // _NS
    nzb = rows_per_tile // _K
    mesh = plsc.VectorSubcoreMesh(core_axis_name="c", subcore_axis_name="s")

    @functools.partial(
        pl.kernel,
        out_type=jax.ShapeDtypeStruct((_NC, n_pad, d), jnp.float32),
        mesh=mesh,
        scratch_types=[
            pltpu.VMEM((2, _K), jnp.int32),
            pltpu.VMEM((2, _K), jnp.int32),
            pltpu.VMEM((2, _K), jnp.float32),
            pltpu.VMEM((2, _K, d), jnp.float32),
            pltpu.VMEM_SHARED((n_pad, d), jnp.float32),
            (pltpu.SemaphoreType.DMA, pltpu.SemaphoreType.DMA),
            (pltpu.SemaphoreType.DMA, pltpu.SemaphoreType.DMA),
        ],
    )
    def spmm(xt_hbm, col_hbm, row_hbm, w_hbm, out_hbm, cbuf, rbuf, wbuf,
             gbuf, acc, sems, esems):
        cid = lax.axis_index("c")
        sid = lax.axis_index("s")
        nch = jnp.where(cid == 0, _NCH0, _NCH1)
        base = jnp.where(cid == 0, sid * _NCH0, _NS * _NCH0 + sid * _NCH1)

        zv = jnp.zeros((_L,), jnp.float32)

        def _zrow(i, carry):
            for l in range(d // _L):
                gbuf[0, i, pl.ds(l * _L, _L)] = zv
            return carry

        lax.fori_loop(0, _K, _zrow, 0)
        for b in range(nzb):
            pltpu.sync_copy(
                gbuf.at[0], acc.at[pl.ds(sid * rows_per_tile + b * _K, _K)])
        plsc.subcore_barrier()

        def _estart(ch, b):
            off = (base + ch) * _K
            pltpu.async_copy(col_hbm.at[pl.ds(off, _K)], cbuf.at[b],
                             esems[b])
            pltpu.async_copy(row_hbm.at[pl.ds(off, _K)], rbuf.at[b],
                             esems[b])
            pltpu.async_copy(w_hbm.at[pl.ds(off, _K)], wbuf.at[b],
                             esems[b])

        def _ewait(b):
            pltpu.make_async_copy(
                col_hbm.at[pl.ds(0, _K)], cbuf.at[b], esems[b]).wait()
            pltpu.make_async_copy(
                row_hbm.at[pl.ds(0, _K)], rbuf.at[b], esems[b]).wait()
            pltpu.make_async_copy(
                w_hbm.at[pl.ds(0, _K)], wbuf.at[b], esems[b]).wait()

        def _gstart(b):
            pltpu.async_copy(
                xt_hbm.at[cbuf.at[b]], gbuf.at[b], sems[b])

        def _gwait(b):
            pltpu.make_async_copy(
                xt_hbm.at[pl.ds(0, _K)], gbuf.at[b], sems[b]).wait()

        def _process(b):
            def _group(g, c2):
                wvec = wbuf[b, pl.ds(g * _L, _L)]
                for j in range(_L):
                    w = wvec[j]
                    row = g * _L + j
                    for l in range(d // _L):
                        sl = pl.ds(l * _L, _L)
                        gbuf[b, row, sl] = gbuf[b, row, sl] * w
                return c2

            lax.fori_loop(0, _K // _L, _group, 0)
            pltpu.sync_copy(gbuf.at[b], acc.at[rbuf.at[b]], add=True)

        _estart(0, 0)
        _ewait(0)
        _gstart(0)
        _estart(1, 1)
        _ewait(1)
        _gstart(1)

        def _pair(p, carry):
            ch0 = 2 * p
            _gwait(0)
            _process(0)
            _estart(ch0 + 2, 0)
            _ewait(0)
            _gstart(0)
            _gwait(1)
            _process(1)

            @pl.when(ch0 + 3 < nch)
            def _():
                _estart(ch0 + 3, 1)
                _ewait(1)
                _gstart(1)

            return carry

        lax.fori_loop(0, (nch - 1) // 2, _pair, 0)
        _gwait(0)
        _process(0)
        plsc.subcore_barrier()
        for b in range(nzb):
            off = sid * rows_per_tile + b * _K
            pltpu.sync_copy(acc.at[pl.ds(off, _K)],
                            out_hbm.at[cid, pl.ds(off, _K)])

    return spmm(*args)


def kernel(x, edge_index, edge_weight):
    n, d = x.shape
    e = edge_weight.shape[0]
    n_pad = -(-n // (_NS * _K)) * (_NS * _K)
    etot = _NS * (_NCH0 + _NCH1) * _K

    blk = 1000
    grid = n // blk
    xt = pl.pallas_call(
        _pre_body,
        grid=(grid,),
        in_specs=[pl.BlockSpec((blk, d), lambda i: (i, 0))],
        out_specs=pl.BlockSpec((blk, d), lambda i: (i, 0)),
        out_shape=jax.ShapeDtypeStruct((n, d), jnp.float32),
    )(x)

    pad = etot - e
    colp = jnp.pad(edge_index[1], (0, pad))
    rowp = jnp.pad(edge_index[0], (0, pad))
    wp = jnp.pad(edge_weight, (0, pad))

    partials = _sc_spmm((xt, colp, rowp, wp), n_pad, d)

    out = pl.pallas_call(
        _post_body,
        grid=(grid,),
        in_specs=[pl.BlockSpec((1, blk, d), lambda i: (0, i, 0)),
                  pl.BlockSpec((1, blk, d), lambda i: (1, i, 0))],
        out_specs=pl.BlockSpec((blk, d), lambda i: (i, 0)),
        out_shape=jax.ShapeDtypeStruct((n, d), jnp.float32),
    )(partials, partials)
    return out


# speedup vs baseline: 1.1947x; 1.1947x over previous
import functools

import jax
import jax.numpy as jnp
from jax import lax
from jax.experimental import pallas as pl
from jax.experimental.pallas import tpu as pltpu
from jax.experimental.pallas import tpu_sc as plsc

_C = 1.0
_MAX_NORM = 1e6
_EPS = 1e-15
_BALL_EPS = 4e-3
_HALF = 64

_NC = 2
_NS = 16
_NW = _NC * _NS
_L = 16
_K = 128

_NCH0 = 115
_NCH1 = 43


def _artanh(x):
    x = jnp.clip(x, -1.0 + 1e-7, 1.0 - 1e-7)
    return 0.5 * (jnp.log1p(x) - jnp.log1p(-x))


def _pre_body(x_ref, o_ref):
    v = x_ref[...]
    outs = []
    for lo in (0, _HALF):
        s = v[:, lo:lo + _HALF]
        n = jnp.maximum(jnp.sqrt(jnp.sum(s * s, axis=1, keepdims=True)), _EPS)
        outs.append(s * (_artanh(n) / n))
    o_ref[...] = jnp.concatenate(outs, axis=1)


def _post_body(p0_ref, p1_ref, o_ref):
    s = jnp.minimum(p0_ref[0] + p1_ref[0], _MAX_NORM)
    outs = []
    for lo in (0, _HALF):
        u = s[:, lo:lo + _HALF]
        n = jnp.maximum(jnp.sqrt(jnp.sum(u * u, axis=1, keepdims=True)), _EPS)
        y = u * (jnp.tanh(n) / n)
        yn = jnp.maximum(jnp.sqrt(jnp.sum(y * y, axis=1, keepdims=True)), _EPS)
        maxn = 1.0 - _BALL_EPS
        outs.append(jnp.where(yn > maxn, y / yn * maxn, y))
    o_ref[...] = jnp.concatenate(outs, axis=1)


@functools.partial(jax.jit, static_argnums=(1, 2))
def _sc_spmm(args, n_pad, d):
    rows_per_tile = n_pad // _NS
    nzb = rows_per_tile // _K
    mesh = plsc.VectorSubcoreMesh(core_axis_name="c", subcore_axis_name="s")

    @functools.partial(
        pl.kernel,
        out_type=jax.ShapeDtypeStruct((_NC, n_pad, d), jnp.float32),
        mesh=mesh,
        compiler_params=pltpu.CompilerParams(use_tc_tiling_on_sc=False),
        scratch_types=[
            pltpu.VMEM((2, _K), jnp.int32),
            pltpu.VMEM((2, _K), jnp.int32),
            pltpu.VMEM((2, _K), jnp.float32),
            pltpu.VMEM((2, _K, d), jnp.float32),
            pltpu.VMEM_SHARED((n_pad, d), jnp.float32),
            (pltpu.SemaphoreType.DMA, pltpu.SemaphoreType.DMA),
            (pltpu.SemaphoreType.DMA, pltpu.SemaphoreType.DMA),
        ],
    )
    def spmm(xt_hbm, col_hbm, row_hbm, w_hbm, out_hbm, cbuf, rbuf, wbuf,
             gbuf, acc, sems, esems):
        cid = lax.axis_index("c")
        sid = lax.axis_index("s")
        nch = jnp.where(cid == 0, _NCH0, _NCH1)
        base = jnp.where(cid == 0, sid * _NCH0, _NS * _NCH0 + sid * _NCH1)

        zv = jnp.zeros((_L,), jnp.float32)

        def _zrow(i, carry):
            for l in range(d // _L):
                gbuf[0, i, pl.ds(l * _L, _L)] = zv
            return carry

        lax.fori_loop(0, _K, _zrow, 0)
        for b in range(nzb):
            pltpu.sync_copy(
                gbuf.at[0], acc.at[pl.ds(sid * rows_per_tile + b * _K, _K)])
        plsc.subcore_barrier()

        def _estart(ch, b):
            off = (base + ch) * _K
            pltpu.async_copy(col_hbm.at[pl.ds(off, _K)], cbuf.at[b],
                             esems[b])
            pltpu.async_copy(row_hbm.at[pl.ds(off, _K)], rbuf.at[b],
                             esems[b])
            pltpu.async_copy(w_hbm.at[pl.ds(off, _K)], wbuf.at[b],
                             esems[b])

        def _ewait(b):
            pltpu.make_async_copy(
                col_hbm.at[pl.ds(0, _K)], cbuf.at[b], esems[b]).wait()
            pltpu.make_async_copy(
                row_hbm.at[pl.ds(0, _K)], rbuf.at[b], esems[b]).wait()
            pltpu.make_async_copy(
                w_hbm.at[pl.ds(0, _K)], wbuf.at[b], esems[b]).wait()

        def _gstart(b):
            pltpu.async_copy(
                xt_hbm.at[cbuf.at[b]], gbuf.at[b], sems[b])

        def _gwait(b):
            pltpu.make_async_copy(
                xt_hbm.at[pl.ds(0, _K)], gbuf.at[b], sems[b]).wait()

        def _process(b):
            def _group(g, c2):
                wvec = wbuf[b, pl.ds(g * _L, _L)]
                for j in range(_L):
                    w = wvec[j]
                    row = g * _L + j
                    for l in range(d // _L):
                        sl = pl.ds(l * _L, _L)
                        gbuf[b, row, sl] = gbuf[b, row, sl] * w
                return c2

            lax.fori_loop(0, _K // _L, _group, 0)
            pltpu.sync_copy(gbuf.at[b], acc.at[rbuf.at[b]], add=True)

        _estart(0, 0)
        _ewait(0)
        _gstart(0)
        _estart(1, 1)
        _ewait(1)
        _gstart(1)

        def _pair(p, carry):
            ch0 = 2 * p
            _gwait(0)
            _process(0)
            _estart(ch0 + 2, 0)
            _ewait(0)
            _gstart(0)
            _gwait(1)
            _process(1)

            @pl.when(ch0 + 3 < nch)
            def _():
                _estart(ch0 + 3, 1)
                _ewait(1)
                _gstart(1)

            return carry

        lax.fori_loop(0, (nch - 1) // 2, _pair, 0)
        _gwait(0)
        _process(0)
        plsc.subcore_barrier()
        for b in range(nzb):
            off = sid * rows_per_tile + b * _K
            pltpu.sync_copy(acc.at[pl.ds(off, _K)],
                            out_hbm.at[cid, pl.ds(off, _K)])

    return spmm(*args)


def kernel(x, edge_index, edge_weight):
    n, d = x.shape
    e = edge_weight.shape[0]
    n_pad = -(-n // (_NS * _K)) * (_NS * _K)
    etot = _NS * (_NCH0 + _NCH1) * _K

    blk = 1000
    grid = n // blk
    xt = pl.pallas_call(
        _pre_body,
        grid=(grid,),
        in_specs=[pl.BlockSpec((blk, d), lambda i: (i, 0))],
        out_specs=pl.BlockSpec((blk, d), lambda i: (i, 0)),
        out_shape=jax.ShapeDtypeStruct((n, d), jnp.float32),
    )(x)

    pad = etot - e
    colp = jnp.pad(edge_index[1], (0, pad))
    rowp = jnp.pad(edge_index[0], (0, pad))
    wp = jnp.pad(edge_weight, (0, pad))

    partials = _sc_spmm((xt, colp, rowp, wp), n_pad, d)

    out = pl.pallas_call(
        _post_body,
        grid=(grid,),
        in_specs=[pl.BlockSpec((1, blk, d), lambda i: (0, i, 0)),
                  pl.BlockSpec((1, blk, d), lambda i: (1, i, 0))],
        out_specs=pl.BlockSpec((blk, d), lambda i: (i, 0)),
        out_shape=jax.ShapeDtypeStruct((n, d), jnp.float32),
    )(partials, partials)
    return out
